# baseline (speedup 1.0000x reference)
import functools

import jax
import jax.numpy as jnp
from jax import lax
from jax.experimental import pallas as pl
from jax.experimental.pallas import tpu as pltpu
from jax.experimental.pallas import tpu_sc as plsc

N = 10000
H = 128
E = 320000
NC = 2
NS = 16
NW = NC * NS
CH = 64
CPW = 160
E2 = NW * CPW * CH
EB = E2 // CH
IB = 16
NIB = CPW // IB
NA = N + 16
DT = 10
DB = N // DT
DC = 40
NDC = DB // DC
K = 5
PRIME = 3


def _seg_sum_builder(with_counts):
  mesh = plsc.VectorSubcoreMesh(core_axis_name="c", subcore_axis_name="s")
  n_out = 4 if with_counts else 2
  out_type = [jax.ShapeDtypeStruct((NC, N, H), jnp.float32)] * n_out
  scratch = [
      pltpu.VMEM((IB, CH), jnp.int32),
      pltpu.VMEM((IB, CH), jnp.int32),
      pltpu.VMEM((DC, H), jnp.float32),
      pltpu.VMEM_SHARED((NA, H), jnp.float32),
  ] + [pltpu.VMEM((CH, H), jnp.float32) for _ in range(K)] + [
      pltpu.SemaphoreType.DMA for _ in range(2 * K)]

  def body(pu, pi, ones_tbl, ei_ui, ei_iu, *refs):
    outs = refs[:n_out]
    ridx, cidx, stage, acc = refs[n_out:n_out + 4]
    bufs = refs[n_out + 4:n_out + 4 + K]
    gsem = refs[n_out + 4 + K:n_out + 4 + 2 * K]
    ssem = refs[n_out + 4 + 2 * K:]
    cid = lax.axis_index("c")
    sid = lax.axis_index("s")
    wid = cid * NS + sid
    c0 = wid * CPW

    def run_dir(p_hbm, ei_hbm, out_hbm, counts):
      def zstage(k, carry):
        stage[k // (H // 16), pl.ds((k % (H // 16)) * 16, 16)] = (
            jnp.zeros((16,), jnp.float32))
        return carry
      lax.fori_loop(0, DC * (H // 16), zstage, 0)

      @pl.when(sid < DT)
      def _():
        for k in range(NDC):
          pltpu.sync_copy(stage, acc.at[pl.ds(sid * DB + k * DC, DC)])
      plsc.subcore_barrier()

      if counts:
        def zridx(k, carry):
          ridx[0, pl.ds(k * 16, 16)] = jnp.zeros((16,), jnp.int32)
          return carry
        lax.fori_loop(0, CH // 16, zridx, 0)
        pltpu.async_copy(p_hbm.at[ridx.at[0]], bufs[0], gsem[0]).wait()

        def cblock(b, carry):
          pltpu.sync_copy(ei_hbm.at[1, pl.ds(c0 + b * IB, IB)], cidx)
          ds_ = [pltpu.async_copy(bufs[0], acc.at[cidx.at[j]], add=True,
                                  sem=ssem[0]) for j in range(IB)]
          for d in ds_:
            d.wait()
          return carry
        lax.fori_loop(0, NIB, cblock, 0)
      else:
        def block(b, carry):
          pltpu.sync_copy(ei_hbm.at[0, pl.ds(c0 + b * IB, IB)], ridx)
          pltpu.sync_copy(ei_hbm.at[1, pl.ds(c0 + b * IB, IB)], cidx)
          gd = {}
          sd = {}
          for k in range(PRIME):
            gd[k] = pltpu.async_copy(
                p_hbm.at[ridx.at[k]], bufs[k % K], gsem[k % K])
          waited = -1
          for j in range(IB):
            gd[j].wait()
            sd[j] = pltpu.async_copy(
                bufs[j % K], acc.at[cidx.at[j]], add=True, sem=ssem[j % K])
            nxt = j + PRIME
            if nxt < IB:
              if nxt - K >= 0:
                sd[nxt - K].wait()
                waited = nxt - K
              gd[nxt] = pltpu.async_copy(
                  p_hbm.at[ridx.at[nxt]], bufs[nxt % K], gsem[nxt % K])
          for j in range(waited + 1, IB):
            sd[j].wait()
          return carry
        lax.fori_loop(0, NIB, block, 0)
      plsc.subcore_barrier()

      @pl.when(sid < DT)
      def _():
        for k in range(NDC):
          r0 = sid * DB + k * DC
          pltpu.sync_copy(acc.at[pl.ds(r0, DC)], stage)
          pltpu.sync_copy(stage, out_hbm.at[cid, pl.ds(r0, DC)])
      plsc.subcore_barrier()

    run_dir(pu, ei_ui, outs[0], False)
    run_dir(pi, ei_iu, outs[1], False)
    if with_counts:
      run_dir(ones_tbl, ei_ui, outs[2], True)
      run_dir(ones_tbl, ei_iu, outs[3], True)

  return functools.partial(
      pl.kernel, body, out_type=out_type, mesh=mesh, scratch_types=scratch)


def _mm(a, b):
  return jnp.dot(a, b, preferred_element_type=jnp.float32)


def _relu(x):
  return jnp.maximum(x, 0.0)


def _prologue_side_body(x, w, b, wl, h_o, p_o):
  h = _relu(_mm(x[...], w[...]) + b[...])
  h_o[...] = h
  p_o[...] = _mm(h, wl[...])


def _bn_relu(z, g_r, b_r):
  m = jnp.mean(z, axis=0, keepdims=True)
  v = jnp.mean((z - m) * (z - m), axis=0, keepdims=True)
  return _relu((z - m) / jnp.sqrt(v + 1e-5) * g_r[...] + b_r[...])


def _agg(s_r, ct_r):
  sf = s_r[...]
  cf = ct_r[...]
  cnt = jnp.maximum(cf[0, :N, 0:1] + cf[1, :N, 0:1], 1.0)
  return (sf[0, :N] + sf[1, :N]) / cnt


def _combine_side_body(s_r, ct_r, h_r, wr_r, bl_r, g_r, b_r, wl1_r, h_o, p_o):
  z = _agg(s_r, ct_r) + bl_r[...] + _mm(h_r[...], wr_r[...])
  n = _bn_relu(z, g_r, b_r)
  h_o[...] = n
  p_o[...] = _mm(n, wl1_r[...])


def _final_side_body(s_r, ct_r, h_r, wr_r, bl_r, g_r, b_r, wo_r, bo_r, out_o):
  z = _agg(s_r, ct_r) + bl_r[...] + _mm(h_r[...], wr_r[...])
  n = _bn_relu(z, g_r, b_r)
  out_o[...] = _mm(h_r[...] + n, wo_r[...]) + bo_r[...]


def _tc_call(body, n_out):
  return pl.pallas_call(
      body, out_shape=[jax.ShapeDtypeStruct((N, H), jnp.float32)] * n_out)


def _pad_edges(ei):
  pad = E2 - E
  pad_block = jnp.concatenate(
      [jnp.zeros((1, pad), jnp.int32), jnp.full((1, pad), N, jnp.int32)])
  return jnp.concatenate([ei, pad_block], axis=1).reshape(2, EB, CH)


def kernel(x_user, x_item, edge_index_user_buys_item,
           edge_index_item_bought_by_user, params):
  p = params
  l0, l1 = p['layers']
  r = lambda v: v.reshape(1, -1)

  ei_ui = _pad_edges(edge_index_user_buys_item)
  ei_iu = _pad_edges(edge_index_item_bought_by_user)
  ones_tbl = jnp.ones((8, H), jnp.float32)

  hu0, pu0 = _tc_call(_prologue_side_body, 2)(
      x_user, p['in_proj']['user']['W'], r(p['in_proj']['user']['b']),
      l0['ui']['Wl'])
  hi0, pi0 = _tc_call(_prologue_side_body, 2)(
      x_item, p['in_proj']['item']['W'], r(p['in_proj']['item']['b']),
      l0['iu']['Wl'])

  sI0, sU0, cI, cU = _seg_sum_builder(True)()(
      pu0, pi0, ones_tbl, ei_ui, ei_iu)

  hi1, pi1 = _tc_call(_combine_side_body, 2)(
      sI0, cI, hi0, l0['ui']['Wr'], r(l0['ui']['bl']),
      r(l0['bn_item']['g']), r(l0['bn_item']['b']), l1['iu']['Wl'])
  hu1, pu1 = _tc_call(_combine_side_body, 2)(
      sU0, cU, hu0, l0['iu']['Wr'], r(l0['iu']['bl']),
      r(l0['bn_user']['g']), r(l0['bn_user']['b']), l1['ui']['Wl'])

  sI1, sU1 = _seg_sum_builder(False)()(pu1, pi1, ones_tbl, ei_ui, ei_iu)

  out_item = _tc_call(_final_side_body, 1)(
      sI1, cI, hi1, l1['ui']['Wr'], r(l1['ui']['bl']),
      r(l1['bn_item']['g']), r(l1['bn_item']['b']),
      p['out_proj']['item']['W'], r(p['out_proj']['item']['b']))[0]
  out_user = _tc_call(_final_side_body, 1)(
      sU1, cU, hu1, l1['iu']['Wr'], r(l1['iu']['bl']),
      r(l1['bn_user']['g']), r(l1['bn_user']['b']),
      p['out_proj']['user']['W'], r(p['out_proj']['user']['b']))[0]
  return (out_user, out_item)


# speedup vs baseline: 2.0980x; 2.0980x over previous
import functools

import jax
import jax.numpy as jnp
import numpy as np
from jax import lax
from jax.experimental import pallas as pl
from jax.experimental.pallas import tpu as pltpu
from jax.experimental.pallas import tpu_sc as plsc

N = 10000
H = 128
E = 320000
NC = 2
NS = 16
NW = NC * NS
CH = 128
CPW = 80
E2 = NW * CPW * CH
EB = E2 // CH
IB = 16
NIB = CPW // IB
DR = 112
NA = N + DR
DT = 10
DB = N // DT
DC = 40
NDC = DB // DC
K = 2
PRIME = 1


def _seg_sum_builder(with_counts):
  mesh = plsc.VectorSubcoreMesh(core_axis_name="c", subcore_axis_name="s")
  n_out = 4 if with_counts else 2
  out_type = [jax.ShapeDtypeStruct((NC, N, H), jnp.float32)] * n_out
  scratch = [
      pltpu.VMEM((IB, CH), jnp.int32),
      pltpu.VMEM((IB, CH), jnp.int32),
      pltpu.VMEM((DC, H), jnp.float32),
      pltpu.VMEM_SHARED((NA, H), jnp.float32),
  ] + [pltpu.VMEM((CH, H), jnp.float32) for _ in range(K)] + [
      pltpu.SemaphoreType.DMA for _ in range(2 * K)]

  def body(pu, pi, ones_tbl, ei_ui, ei_iu, *refs):
    outs = refs[:n_out]
    ridx, cidx, stage, acc = refs[n_out:n_out + 4]
    bufs = refs[n_out + 4:n_out + 4 + K]
    gsem = refs[n_out + 4 + K:n_out + 4 + 2 * K]
    ssem = refs[n_out + 4 + 2 * K:]
    cid = lax.axis_index("c")
    sid = lax.axis_index("s")
    wid = cid * NS + sid
    c0 = wid * CPW

    def run_dir(p_hbm, ei_hbm, out_hbm, counts):
      def zstage(k, carry):
        stage[k // (H // 16), pl.ds((k % (H // 16)) * 16, 16)] = (
            jnp.zeros((16,), jnp.float32))
        return carry
      lax.fori_loop(0, DC * (H // 16), zstage, 0)

      @pl.when(sid < DT)
      def _():
        for k in range(NDC):
          pltpu.sync_copy(stage, acc.at[pl.ds(sid * DB + k * DC, DC)])
      plsc.subcore_barrier()

      if counts:
        def zridx(k, carry):
          ridx[0, pl.ds(k * 16, 16)] = jnp.zeros((16,), jnp.int32)
          return carry
        lax.fori_loop(0, CH // 16, zridx, 0)
        pltpu.async_copy(p_hbm.at[ridx.at[0]], bufs[0], gsem[0]).wait()

        def cblock(b, carry):
          pltpu.sync_copy(ei_hbm.at[1, pl.ds(c0 + b * IB, IB)], cidx)
          ds_ = [pltpu.async_copy(bufs[0], acc.at[cidx.at[j]], add=True,
                                  sem=ssem[j % K]) for j in range(IB)]
          for d in ds_:
            d.wait()
          return carry
        lax.fori_loop(0, NIB, cblock, 0)
      else:
        def block(b, carry):
          pltpu.sync_copy(ei_hbm.at[0, pl.ds(c0 + b * IB, IB)], ridx)
          pltpu.sync_copy(ei_hbm.at[1, pl.ds(c0 + b * IB, IB)], cidx)
          gd = {}
          sd = {}
          for k in range(PRIME):
            gd[k] = pltpu.async_copy(
                p_hbm.at[ridx.at[k]], bufs[k % K], gsem[k % K])
          waited = -1
          for j in range(IB):
            gd[j].wait()
            sd[j] = pltpu.async_copy(
                bufs[j % K], acc.at[cidx.at[j]], add=True, sem=ssem[j % K])
            nxt = j + PRIME
            if nxt < IB:
              if nxt - K >= 0:
                sd[nxt - K].wait()
                waited = nxt - K
              gd[nxt] = pltpu.async_copy(
                  p_hbm.at[ridx.at[nxt]], bufs[nxt % K], gsem[nxt % K])
          for j in range(waited + 1, IB):
            sd[j].wait()
          return carry
        lax.fori_loop(0, NIB, block, 0)
      plsc.subcore_barrier()

      @pl.when(sid < DT)
      def _():
        for k in range(NDC):
          r0 = sid * DB + k * DC
          pltpu.sync_copy(acc.at[pl.ds(r0, DC)], stage)
          pltpu.sync_copy(stage, out_hbm.at[cid, pl.ds(r0, DC)])
      plsc.subcore_barrier()

    run_dir(pu, ei_ui, outs[0], False)
    run_dir(pi, ei_iu, outs[1], False)
    if with_counts:
      run_dir(ones_tbl, ei_ui, outs[2], True)
      run_dir(ones_tbl, ei_iu, outs[3], True)

  return functools.partial(
      pl.kernel, body, out_type=out_type, mesh=mesh, scratch_types=scratch)


def _mm(a, b):
  return jnp.dot(a, b, preferred_element_type=jnp.float32)


def _relu(x):
  return jnp.maximum(x, 0.0)


def _prologue_side_body(x, w, b, wl, h_o, p_o):
  h = _relu(_mm(x[...], w[...]) + b[...])
  h_o[...] = h
  p_o[...] = _mm(h, wl[...])


def _bn_relu(z, g_r, b_r):
  m = jnp.mean(z, axis=0, keepdims=True)
  v = jnp.mean((z - m) * (z - m), axis=0, keepdims=True)
  return _relu((z - m) / jnp.sqrt(v + 1e-5) * g_r[...] + b_r[...])


def _agg(s_r, ct_r):
  sf = s_r[...]
  cf = ct_r[...]
  cnt = jnp.maximum(cf[0, :N, 0:1] + cf[1, :N, 0:1], 1.0)
  return (sf[0, :N] + sf[1, :N]) / cnt


def _combine_side_body(s_r, ct_r, h_r, wr_r, bl_r, g_r, b_r, wl1_r, h_o, p_o):
  z = _agg(s_r, ct_r) + bl_r[...] + _mm(h_r[...], wr_r[...])
  n = _bn_relu(z, g_r, b_r)
  h_o[...] = n
  p_o[...] = _mm(n, wl1_r[...])


def _final_side_body(s_r, ct_r, h_r, wr_r, bl_r, g_r, b_r, wo_r, bo_r, out_o):
  z = _agg(s_r, ct_r) + bl_r[...] + _mm(h_r[...], wr_r[...])
  n = _bn_relu(z, g_r, b_r)
  out_o[...] = _mm(h_r[...] + n, wo_r[...]) + bo_r[...]


def _tc_call(body, n_out):
  return pl.pallas_call(
      body, out_shape=[jax.ShapeDtypeStruct((N, H), jnp.float32)] * n_out)


def _pad_edges(ei):
  i = np.arange(E2 - E)
  pad_block = jnp.asarray(
      np.stack([i % N, N + i % DR]).astype(np.int32))
  return jnp.concatenate([ei, pad_block], axis=1).reshape(2, EB, CH)


def kernel(x_user, x_item, edge_index_user_buys_item,
           edge_index_item_bought_by_user, params):
  p = params
  l0, l1 = p['layers']
  r = lambda v: v.reshape(1, -1)

  ei_ui = _pad_edges(edge_index_user_buys_item)
  ei_iu = _pad_edges(edge_index_item_bought_by_user)
  ones_tbl = jnp.ones((8, H), jnp.float32)

  hu0, pu0 = _tc_call(_prologue_side_body, 2)(
      x_user, p['in_proj']['user']['W'], r(p['in_proj']['user']['b']),
      l0['ui']['Wl'])
  hi0, pi0 = _tc_call(_prologue_side_body, 2)(
      x_item, p['in_proj']['item']['W'], r(p['in_proj']['item']['b']),
      l0['iu']['Wl'])

  sI0, sU0, cI, cU = _seg_sum_builder(True)()(
      pu0, pi0, ones_tbl, ei_ui, ei_iu)

  hi1, pi1 = _tc_call(_combine_side_body, 2)(
      sI0, cI, hi0, l0['ui']['Wr'], r(l0['ui']['bl']),
      r(l0['bn_item']['g']), r(l0['bn_item']['b']), l1['iu']['Wl'])
  hu1, pu1 = _tc_call(_combine_side_body, 2)(
      sU0, cU, hu0, l0['iu']['Wr'], r(l0['iu']['bl']),
      r(l0['bn_user']['g']), r(l0['bn_user']['b']), l1['ui']['Wl'])

  sI1, sU1 = _seg_sum_builder(False)()(pu1, pi1, ones_tbl, ei_ui, ei_iu)

  out_item = _tc_call(_final_side_body, 1)(
      sI1, cI, hi1, l1['ui']['Wr'], r(l1['ui']['bl']),
      r(l1['bn_item']['g']), r(l1['bn_item']['b']),
      p['out_proj']['item']['W'], r(p['out_proj']['item']['b']))[0]
  out_user = _tc_call(_final_side_body, 1)(
      sU1, cU, hu1, l1['iu']['Wr'], r(l1['iu']['bl']),
      r(l1['bn_user']['g']), r(l1['bn_user']['b']),
      p['out_proj']['user']['W'], r(p['out_proj']['user']['b']))[0]
  return (out_user, out_item)


# speedup vs baseline: 2.1840x; 1.0410x over previous
import functools

import jax
import jax.numpy as jnp
import numpy as np
from jax import lax
from jax.experimental import pallas as pl
from jax.experimental.pallas import tpu as pltpu
from jax.experimental.pallas import tpu_sc as plsc

N = 10000
H = 128
E = 320000
NC = 2
NS = 16
NW = NC * NS
CH = 128
CPW = 80
E2 = NW * CPW * CH
EB = E2 // CH
IB = 16
NIB = CPW // IB
DR = 112
NA = N + DR
DT = 10
DB = N // DT
DC = 40
NDC = DB // DC
K = 2
PRIME = 1


def _seg_sum_builder(with_counts):
  mesh = plsc.VectorSubcoreMesh(core_axis_name="c", subcore_axis_name="s")
  n_out = 4 if with_counts else 2
  out_type = [jax.ShapeDtypeStruct((NC, N, H), jnp.float32)] * n_out
  scratch = [
      pltpu.VMEM((IB, CH), jnp.int32),
      pltpu.VMEM((IB, CH), jnp.int32),
      pltpu.VMEM((DC, H), jnp.float32),
      pltpu.VMEM((DC, H), jnp.float32),
      pltpu.VMEM_SHARED((NA, H), jnp.float32),
  ] + [pltpu.VMEM((CH, H), jnp.float32) for _ in range(K)] + [
      pltpu.SemaphoreType.DMA for _ in range(2 * K)]

  def body(pu, pi, ones_tbl, ei_ui, ei_iu, *refs):
    outs = refs[:n_out]
    ridx, cidx, stage, stage2, acc = refs[n_out:n_out + 5]
    bufs = refs[n_out + 5:n_out + 5 + K]
    gsem = refs[n_out + 5 + K:n_out + 5 + 2 * K]
    ssem = refs[n_out + 5 + 2 * K:]
    cid = lax.axis_index("c")
    sid = lax.axis_index("s")
    wid = cid * NS + sid
    c0 = wid * CPW

    def run_dir(p_hbm, ei_hbm, out_hbm, counts):
      def zstage(k, carry):
        stage[k // (H // 16), pl.ds((k % (H // 16)) * 16, 16)] = (
            jnp.zeros((16,), jnp.float32))
        return carry
      lax.fori_loop(0, DC * (H // 16), zstage, 0)

      @pl.when(sid < DT)
      def _():
        zd = [pltpu.async_copy(
            stage, acc.at[pl.ds(sid * DB + k * DC, DC)], gsem[0])
              for k in range(NDC)]
        for d in zd:
          d.wait()
      plsc.subcore_barrier()

      if counts:
        def zridx(k, carry):
          ridx[0, pl.ds(k * 16, 16)] = jnp.zeros((16,), jnp.int32)
          return carry
        lax.fori_loop(0, CH // 16, zridx, 0)
        pltpu.async_copy(p_hbm.at[ridx.at[0]], bufs[0], gsem[0]).wait()

        def cblock(b, carry):
          pltpu.sync_copy(ei_hbm.at[1, pl.ds(c0 + b * IB, IB)], cidx)
          ds_ = [pltpu.async_copy(bufs[0], acc.at[cidx.at[j]], add=True,
                                  sem=ssem[j % K]) for j in range(IB)]
          for d in ds_:
            d.wait()
          return carry
        lax.fori_loop(0, NIB, cblock, 0)
      else:
        def block(b, carry):
          pltpu.sync_copy(ei_hbm.at[0, pl.ds(c0 + b * IB, IB)], ridx)
          pltpu.sync_copy(ei_hbm.at[1, pl.ds(c0 + b * IB, IB)], cidx)
          gd = {}
          sd = {}
          for k in range(PRIME):
            gd[k] = pltpu.async_copy(
                p_hbm.at[ridx.at[k]], bufs[k % K], gsem[k % K])
          waited = -1
          for j in range(IB):
            gd[j].wait()
            sd[j] = pltpu.async_copy(
                bufs[j % K], acc.at[cidx.at[j]], add=True, sem=ssem[j % K])
            nxt = j + PRIME
            if nxt < IB:
              if nxt - K >= 0:
                sd[nxt - K].wait()
                waited = nxt - K
              gd[nxt] = pltpu.async_copy(
                  p_hbm.at[ridx.at[nxt]], bufs[nxt % K], gsem[nxt % K])
          for j in range(waited + 1, IB):
            sd[j].wait()
          return carry
        lax.fori_loop(0, NIB, block, 0)
      plsc.subcore_barrier()

      @pl.when(sid < DT)
      def _():
        stages = (stage, stage2)
        d2 = {}
        for k in range(NDC):
          r0 = sid * DB + k * DC
          if k >= 2:
            d2[k - 2].wait()
          pltpu.async_copy(
              acc.at[pl.ds(r0, DC)], stages[k % 2], gsem[k % 2]).wait()
          d2[k] = pltpu.async_copy(
              stages[k % 2], out_hbm.at[cid, pl.ds(r0, DC)], ssem[k % 2])
        d2[NDC - 2].wait()
        d2[NDC - 1].wait()
      plsc.subcore_barrier()

    run_dir(pu, ei_ui, outs[0], False)
    run_dir(pi, ei_iu, outs[1], False)
    if with_counts:
      run_dir(ones_tbl, ei_ui, outs[2], True)
      run_dir(ones_tbl, ei_iu, outs[3], True)

  return functools.partial(
      pl.kernel, body, out_type=out_type, mesh=mesh, scratch_types=scratch)


def _mm(a, b):
  return jnp.dot(a, b, preferred_element_type=jnp.float32)


def _relu(x):
  return jnp.maximum(x, 0.0)


def _prologue_side_body(x, w, b, wl, h_o, p_o):
  h = _relu(_mm(x[...], w[...]) + b[...])
  h_o[...] = h
  p_o[...] = _mm(h, wl[...])


def _bn_relu(z, g_r, b_r):
  m = jnp.mean(z, axis=0, keepdims=True)
  v = jnp.mean((z - m) * (z - m), axis=0, keepdims=True)
  return _relu((z - m) / jnp.sqrt(v + 1e-5) * g_r[...] + b_r[...])


def _agg(s_r, ct_r):
  sf = s_r[...]
  cf = ct_r[...]
  cnt = jnp.maximum(cf[0, :N, 0:1] + cf[1, :N, 0:1], 1.0)
  return (sf[0, :N] + sf[1, :N]) / cnt


def _combine_side_body(s_r, ct_r, h_r, wr_r, bl_r, g_r, b_r, wl1_r, h_o, p_o):
  z = _agg(s_r, ct_r) + bl_r[...] + _mm(h_r[...], wr_r[...])
  n = _bn_relu(z, g_r, b_r)
  h_o[...] = n
  p_o[...] = _mm(n, wl1_r[...])


def _final_side_body(s_r, ct_r, h_r, wr_r, bl_r, g_r, b_r, wo_r, bo_r, out_o):
  z = _agg(s_r, ct_r) + bl_r[...] + _mm(h_r[...], wr_r[...])
  n = _bn_relu(z, g_r, b_r)
  out_o[...] = _mm(h_r[...] + n, wo_r[...]) + bo_r[...]


def _tc_call(body, n_out):
  return pl.pallas_call(
      body, out_shape=[jax.ShapeDtypeStruct((N, H), jnp.float32)] * n_out)


def _pad_edges(ei):
  i = np.arange(E2 - E)
  pad_block = jnp.asarray(
      np.stack([i % N, N + i % DR]).astype(np.int32))
  return jnp.concatenate([ei, pad_block], axis=1).reshape(2, EB, CH)


def kernel(x_user, x_item, edge_index_user_buys_item,
           edge_index_item_bought_by_user, params):
  p = params
  l0, l1 = p['layers']
  r = lambda v: v.reshape(1, -1)

  ei_ui = _pad_edges(edge_index_user_buys_item)
  ei_iu = _pad_edges(edge_index_item_bought_by_user)
  ones_tbl = jnp.ones((8, H), jnp.float32)

  hu0, pu0 = _tc_call(_prologue_side_body, 2)(
      x_user, p['in_proj']['user']['W'], r(p['in_proj']['user']['b']),
      l0['ui']['Wl'])
  hi0, pi0 = _tc_call(_prologue_side_body, 2)(
      x_item, p['in_proj']['item']['W'], r(p['in_proj']['item']['b']),
      l0['iu']['Wl'])

  sI0, sU0, cI, cU = _seg_sum_builder(True)()(
      pu0, pi0, ones_tbl, ei_ui, ei_iu)

  hi1, pi1 = _tc_call(_combine_side_body, 2)(
      sI0, cI, hi0, l0['ui']['Wr'], r(l0['ui']['bl']),
      r(l0['bn_item']['g']), r(l0['bn_item']['b']), l1['iu']['Wl'])
  hu1, pu1 = _tc_call(_combine_side_body, 2)(
      sU0, cU, hu0, l0['iu']['Wr'], r(l0['iu']['bl']),
      r(l0['bn_user']['g']), r(l0['bn_user']['b']), l1['ui']['Wl'])

  sI1, sU1 = _seg_sum_builder(False)()(pu1, pi1, ones_tbl, ei_ui, ei_iu)

  out_item = _tc_call(_final_side_body, 1)(
      sI1, cI, hi1, l1['ui']['Wr'], r(l1['ui']['bl']),
      r(l1['bn_item']['g']), r(l1['bn_item']['b']),
      p['out_proj']['item']['W'], r(p['out_proj']['item']['b']))[0]
  out_user = _tc_call(_final_side_body, 1)(
      sU1, cU, hu1, l1['iu']['Wr'], r(l1['iu']['bl']),
      r(l1['bn_user']['g']), r(l1['bn_user']['b']),
      p['out_proj']['user']['W'], r(p['out_proj']['user']['b']))[0]
  return (out_user, out_item)
